# baseline (speedup 1.0000x reference)
import functools

import jax
import jax.numpy as jnp
from jax import lax
from jax.experimental import pallas as pl
from jax.experimental.pallas import tpu as pltpu
from jax.experimental.pallas import tpu_sc as plsc

_LANES = 16
_NBUF = 8
_PREF = 4


def _body(rows_per_w, nc, table_hbm, words_hbm, out_hbm, idx_v, rows_v, gsem, ssem):
    hist = words_hbm.shape[1]
    wid = lax.axis_index("s") * nc + lax.axis_index("c")
    base = wid * rows_per_w
    pltpu.sync_copy(words_hbm.at[pl.ds(base, rows_per_w)], idx_v)

    def gather(r, slot):
        return pltpu.make_async_copy(
            table_hbm.at[idx_v.at[r]],
            rows_v.at[slot],
            gsem.at[slot],
        )

    def store(r, slot):
        return pltpu.make_async_copy(
            rows_v.at[slot],
            out_hbm.at[base + r],
            ssem.at[slot],
        )

    offs = []
    o = 0
    while o + _LANES < hist:
        offs.append(o)
        o += _LANES
    offs.append(hist - _LANES)

    for r in range(_PREF):
        gather(r, r % _NBUF).start()

    def step(r, carry):
        slot = lax.rem(r, _NBUF)

        @pl.when(r >= _PREF)
        def _():
            store(r - _PREF, lax.rem(r - _PREF, _NBUF)).wait()

        @pl.when(r + _PREF < rows_per_w)
        def _():
            gather(r + _PREF, lax.rem(r + _PREF, _NBUF)).start()

        gather(r, slot).wait()

        acc = jnp.zeros((_LANES,), jnp.int32)
        for o in offs:
            v = idx_v[r, pl.ds(o, _LANES)]
            acc = acc + (v == 0).astype(jnp.int32)
        zc = jnp.sum(acc)

        @pl.when(zc > 0)
        def _fixup():
            zero = jnp.zeros((_LANES,), jnp.float32)
            sid = jnp.full((_LANES,), slot, jnp.int32)
            for o in offs:
                v = idx_v[r, pl.ds(o, _LANES)]
                m = v == 0
                rid = lax.iota(jnp.int32, _LANES) + o
                for c in range(rows_v.shape[2]):
                    cid = jnp.full((_LANES,), c, jnp.int32)
                    plsc.store_scatter(rows_v, [sid, rid, cid], zero, mask=m)

        store(r, slot).start()
        return carry

    lax.fori_loop(0, rows_per_w, step, 0)

    for r in range(rows_per_w - _PREF, rows_per_w):
        store(r, r % _NBUF).wait()


def kernel(words, table):
    B, H = words.shape
    V, D = table.shape
    info = plsc.get_sparse_core_info()
    nc, ns = info.num_cores, info.num_subcores
    nw = nc * ns
    rows_per_w = B // nw

    mesh = plsc.VectorSubcoreMesh(core_axis_name="c", subcore_axis_name="s")
    run = pl.kernel(
        functools.partial(_body, rows_per_w, nc),
        out_type=jax.ShapeDtypeStruct((B, H, D), jnp.float32),
        mesh=mesh,
        compiler_params=pltpu.CompilerParams(
            use_tc_tiling_on_sc=False, needs_layout_passes=False
        ),
        scratch_types=[
            pltpu.VMEM((rows_per_w, H), jnp.int32),
            pltpu.VMEM((_NBUF, H, D), jnp.float32),
            pltpu.SemaphoreType.DMA((_NBUF,)),
            pltpu.SemaphoreType.DMA((_NBUF,)),
        ],
    )
    return run(table, words.astype(jnp.int32))
